# baseline (speedup 1.0000x reference)
import functools

import jax
import jax.numpy as jnp
from jax import lax
from jax.experimental import pallas as pl
from jax.experimental.pallas import tpu as pltpu
from jax.experimental.pallas import tpu_sc as plsc

N_NODES_C = 10000
N_EDGES_C = 320000
HID_C = 128

NC = 2
NS = 16
NW = NC * NS

K2_CH = 25
K4_CH = 50
K2_ROWS_PER_W = (N_EDGES_C // K2_CH) // NW
K2_INNER = 8
K2_OUTER = K2_ROWS_PER_W // K2_INNER
NODES_PER_TILE = N_NODES_C // NS
CNT_PER_TILE = 624
CNT_TAIL = N_NODES_C - NS * CNT_PER_TILE

K4_INNER = 16
K4_OUTER = (2 * N_EDGES_C // K4_CH) // NW // K4_INNER


def _enc_body(x_ref, w_ref, b_ref, o_ref):
    o_ref[...] = jax.nn.relu(
        jnp.dot(x_ref[...], w_ref[...], preferred_element_type=jnp.float32)
        + b_ref[...]
    )


def _sage_body(agg_ref, cnt_ref, h_ref, wl_ref, bl_ref, wr_ref, wcls_ref, o_ref):
    agg = agg_ref[0] + agg_ref[1]
    cnt = cnt_ref[0] + cnt_ref[1]
    mean = agg / jnp.maximum(cnt, 1.0)[:, None]
    h2 = (
        jnp.dot(mean, wl_ref[...], preferred_element_type=jnp.float32)
        + bl_ref[...]
        + jnp.dot(h_ref[...], wr_ref[...], preferred_element_type=jnp.float32)
    )
    o_ref[...] = jnp.dot(h2, wcls_ref[...], preferred_element_type=jnp.float32) * (1.0 / 3.0)


def _edge_enc_body(ea_ref, we_ref, be_ref, wcls_ref, bcls_ref, o_ref):
    ea = jax.nn.relu(
        jnp.dot(ea_ref[...], we_ref[...], preferred_element_type=jnp.float32)
        + be_ref[...]
    )
    o_ref[...] = (
        jnp.dot(ea, wcls_ref[...], preferred_element_type=jnp.float32) * (1.0 / 3.0)
        + bcls_ref[...]
    )


def _edge_body(g0_ref, g1_ref, ec_ref, o_ref):
    out_d = o_ref.shape[1]
    o_ref[...] = g0_ref[...][:, :out_d] + g1_ref[...][:, :out_d] + ec_ref[...]


def _agg_sc_body(src2d, dst2d, h_hbm, agg_out, cnt_out,
                 idx_sv, idx_dv, rows_v, ones_v, zb2, zb1, cz_v,
                 agg_sh, cnt_sh, sem):
    c = lax.axis_index("c")
    s = lax.axis_index("s")
    wid = s * NC + c

    zv = jnp.zeros((16,), jnp.float32)
    for r in range(16):
        for c8 in range(8):
            zb2[r, pl.ds(c8 * 16, 16)] = zv
    zb1[pl.ds(0, 16)] = zv
    for i in range(4):
        ones_v[pl.ds(i * 16, 16)] = jnp.ones((16,), jnp.float32)

    def zstep(j, carry):
        pltpu.sync_copy(zb2, agg_sh.at[pl.ds(s * CNT_PER_TILE + j * 16, 16)])
        pltpu.sync_copy(zb1, cnt_sh.at[pl.ds(s * CNT_PER_TILE + j * 16, 16)])
        return carry

    lax.fori_loop(0, CNT_PER_TILE // 16, zstep, 0)

    @pl.when(s == NS - 1)
    def _():
        pltpu.sync_copy(zb2, agg_sh.at[pl.ds(NS * CNT_PER_TILE, CNT_TAIL)])
        pltpu.sync_copy(zb1, cnt_sh.at[pl.ds(NS * CNT_PER_TILE, CNT_TAIL)])

    plsc.subcore_barrier()

    row0 = wid * K2_ROWS_PER_W

    def step(i, carry):
        base = row0 + i * K2_INNER
        pltpu.sync_copy(src2d.at[pl.ds(base, K2_INNER)], idx_sv)
        pltpu.sync_copy(dst2d.at[pl.ds(base, K2_INNER)], idx_dv)
        cps = [
            pltpu.async_copy(
                h_hbm.at[idx_sv.at[j]], rows_v.at[pl.ds(j * K2_CH, K2_CH)], sem
            )
            for j in range(K2_INNER)
        ]
        for cp in cps:
            cp.wait()
        for j in range(K2_INNER):
            pltpu.sync_copy(
                rows_v.at[pl.ds(j * K2_CH, K2_CH)], agg_sh.at[idx_dv.at[j]], add=True
            )
            pltpu.sync_copy(ones_v.at[pl.ds(0, K2_CH)], cnt_sh.at[idx_dv.at[j]], add=True)
        return carry

    lax.fori_loop(0, K2_OUTER, step, 0)
    plsc.subcore_barrier()

    chunk = 104
    for part in range(CNT_PER_TILE // chunk):
        slp = pl.ds(s * CNT_PER_TILE + part * chunk, chunk)
        pltpu.sync_copy(agg_sh.at[slp], rows_v.at[pl.ds(0, chunk)])
        pltpu.sync_copy(rows_v.at[pl.ds(0, chunk)], agg_out.at[c, slp])
    pltpu.sync_copy(cnt_sh.at[pl.ds(s * CNT_PER_TILE, CNT_PER_TILE)], cz_v)
    pltpu.sync_copy(
        cz_v, cnt_out.at[pl.ds(c * N_NODES_C + s * CNT_PER_TILE, CNT_PER_TILE)]
    )

    @pl.when(s == NS - 1)
    def _():
        slt = pl.ds(NS * CNT_PER_TILE, CNT_TAIL)
        pltpu.sync_copy(agg_sh.at[slt], rows_v.at[pl.ds(0, CNT_TAIL)])
        pltpu.sync_copy(rows_v.at[pl.ds(0, CNT_TAIL)], agg_out.at[c, slt])
        pltpu.sync_copy(cnt_sh.at[slt], cz_v.at[pl.ds(0, CNT_TAIL)])
        pltpu.sync_copy(
            cz_v.at[pl.ds(0, CNT_TAIL)],
            cnt_out.at[pl.ds(c * N_NODES_C + NS * CNT_PER_TILE, CNT_TAIL)],
        )


def _gather_sc_body(eidx2d, pp_hbm, g_out, idx_v, g_v, sem):
    c = lax.axis_index("c")
    s = lax.axis_index("s")
    wid = s * NC + c
    row0 = wid * K4_OUTER * K4_INNER

    def step(i, carry):
        base = row0 + i * K4_INNER
        pltpu.sync_copy(eidx2d.at[pl.ds(base, K4_INNER)], idx_v)
        cps = [
            pltpu.async_copy(
                pp_hbm.at[idx_v.at[j]], g_v.at[pl.ds(j * K4_CH, K4_CH)], sem
            )
            for j in range(K4_INNER)
        ]
        for cp in cps:
            cp.wait()
        pltpu.sync_copy(g_v, g_out.at[pl.ds(base * K4_CH, K4_INNER * K4_CH)])
        return carry

    lax.fori_loop(0, K4_OUTER, step, 0)


def kernel(x, edge_index, edge_attr, W_node, b_node, W_edge, b_edge,
           W_l, b_l, W_r, W_cls, b_cls):
    n, d_in = x.shape
    e = edge_attr.shape[0]
    hid = W_node.shape[1]
    out_d = W_cls.shape[1]

    h = pl.pallas_call(
        _enc_body,
        out_shape=jax.ShapeDtypeStruct((n, hid), jnp.float32),
    )(x, W_node, b_node.reshape(1, hid))

    eblk = 5000
    ec = pl.pallas_call(
        _edge_enc_body,
        grid=(e // eblk,),
        in_specs=[
            pl.BlockSpec((eblk, edge_attr.shape[1]), lambda i: (i, 0)),
            pl.BlockSpec((edge_attr.shape[1], hid), lambda i: (0, 0)),
            pl.BlockSpec((1, hid), lambda i: (0, 0)),
            pl.BlockSpec((hid, out_d), lambda i: (0, 0)),
            pl.BlockSpec((1, out_d), lambda i: (0, 0)),
        ],
        out_specs=pl.BlockSpec((eblk, out_d), lambda i: (i, 0)),
        out_shape=jax.ShapeDtypeStruct((e, out_d), jnp.float32),
    )(edge_attr, W_edge, b_edge.reshape(1, hid), W_cls, b_cls.reshape(1, out_d))

    src2d = edge_index[0].reshape(e // K2_CH, K2_CH)
    dst2d = edge_index[1].reshape(e // K2_CH, K2_CH)

    agg2, cnt2 = pl.kernel(
        _agg_sc_body,
        out_type=(
            jax.ShapeDtypeStruct((NC, n, hid), jnp.float32),
            jax.ShapeDtypeStruct((NC * n,), jnp.float32),
        ),
        mesh=plsc.VectorSubcoreMesh(core_axis_name="c", subcore_axis_name="s", num_cores=NC, num_subcores=NS),
        scratch_types=[
            pltpu.VMEM((K2_INNER, K2_CH), jnp.int32),
            pltpu.VMEM((K2_INNER, K2_CH), jnp.int32),
            pltpu.VMEM((K2_INNER * K2_CH, hid), jnp.float32),
            pltpu.VMEM((64,), jnp.float32),
            pltpu.VMEM((16, hid), jnp.float32),
            pltpu.VMEM((16,), jnp.float32),
            pltpu.VMEM((CNT_PER_TILE,), jnp.float32),
            pltpu.VMEM_SHARED((n, hid), jnp.float32),
            pltpu.VMEM_SHARED((n,), jnp.float32),
            pltpu.SemaphoreType.DMA,
        ],
    )(src2d, dst2d, h)
    cnt2 = cnt2.reshape(NC, n)

    PPW = 16
    W_cls_p = jnp.pad(W_cls, ((0, 0), (0, PPW - out_d)))
    pp = pl.pallas_call(
        _sage_body,
        out_shape=jax.ShapeDtypeStruct((n, PPW), jnp.float32),
    )(agg2, cnt2, h, W_l, b_l.reshape(1, hid), W_r, W_cls_p)

    eidx2d = edge_index.reshape(2 * e // K4_CH, K4_CH)
    g = pl.kernel(
        _gather_sc_body,
        out_type=jax.ShapeDtypeStruct((2 * e, PPW), jnp.float32),
        mesh=plsc.VectorSubcoreMesh(core_axis_name="c", subcore_axis_name="s", num_cores=NC, num_subcores=NS),
        compiler_params=pltpu.CompilerParams(use_tc_tiling_on_sc=False),
        scratch_types=[
            pltpu.VMEM((K4_INNER, K4_CH), jnp.int32),
            pltpu.VMEM((K4_INNER * K4_CH, PPW), jnp.float32),
            pltpu.SemaphoreType.DMA,
        ],
    )(eidx2d, pp)

    blk = 5000
    out = pl.pallas_call(
        _edge_body,
        grid=(e // blk,),
        in_specs=[
            pl.BlockSpec((blk, PPW), lambda i: (i, 0)),
            pl.BlockSpec((blk, PPW), lambda i: (i, 0)),
            pl.BlockSpec((blk, out_d), lambda i: (i, 0)),
        ],
        out_specs=pl.BlockSpec((blk, out_d), lambda i: (i, 0)),
        out_shape=jax.ShapeDtypeStruct((e, out_d), jnp.float32),
    )(g[:e], g[e:], ec)

    return out


# speedup vs baseline: 1.1173x; 1.1173x over previous
import functools

import jax
import jax.numpy as jnp
from jax import lax
from jax.experimental import pallas as pl
from jax.experimental.pallas import tpu as pltpu
from jax.experimental.pallas import tpu_sc as plsc

N_NODES_C = 10000
N_EDGES_C = 320000
HID_C = 128

NC = 2
NS = 16
NW = NC * NS

K2_CH = 25
K4_CH = 50
K2_ROWS_PER_W = (N_EDGES_C // K2_CH) // NW
K2_INNER = 8
K2_OUTER = K2_ROWS_PER_W // K2_INNER
NODES_PER_TILE = N_NODES_C // NS
CNT_PER_TILE = 624
CNT_TAIL = N_NODES_C - NS * CNT_PER_TILE

K4_INNER = 16
K4_OUTER = (2 * N_EDGES_C // K4_CH) // NW // K4_INNER


def _enc_body(x_ref, w_ref, b_ref, o_ref):
    o_ref[...] = jax.nn.relu(
        jnp.dot(x_ref[...], w_ref[...], preferred_element_type=jnp.float32)
        + b_ref[...]
    )


def _sage_body(agg_ref, cnt_ref, h_ref, wl_ref, bl_ref, wr_ref, wcls_ref, o_ref):
    agg = agg_ref[0] + agg_ref[1]
    cnt = cnt_ref[0] + cnt_ref[1]
    mean = agg / jnp.maximum(cnt, 1.0)[:, None]
    h2 = (
        jnp.dot(mean, wl_ref[...], preferred_element_type=jnp.float32)
        + bl_ref[...]
        + jnp.dot(h_ref[...], wr_ref[...], preferred_element_type=jnp.float32)
    )
    o_ref[...] = jnp.dot(h2, wcls_ref[...], preferred_element_type=jnp.float32) * (1.0 / 3.0)


def _edge_body(g0_ref, g1_ref, ea_ref, we_ref, be_ref, wcls_ref, bcls_ref, o_ref):
    ea = jax.nn.relu(
        jnp.dot(ea_ref[...], we_ref[...], preferred_element_type=jnp.float32)
        + be_ref[...]
    )
    out_d = o_ref.shape[1]
    o_ref[...] = (
        g0_ref[...][:, :out_d]
        + g1_ref[...][:, :out_d]
        + jnp.dot(ea, wcls_ref[...], preferred_element_type=jnp.float32) * (1.0 / 3.0)
        + bcls_ref[...]
    )


def _agg_sc_body(src2d, dst2d, h_hbm, agg_out, cnt_out,
                 idx_sv, idx_dv, rows_v, ones_v, zb2, zb1, cz_v,
                 agg_sh, cnt_sh, sem):
    c = lax.axis_index("c")
    s = lax.axis_index("s")
    wid = s * NC + c

    zv = jnp.zeros((16,), jnp.float32)
    for r in range(16):
        for c8 in range(8):
            zb2[r, pl.ds(c8 * 16, 16)] = zv
    zb1[pl.ds(0, 16)] = zv
    for i in range(4):
        ones_v[pl.ds(i * 16, 16)] = jnp.ones((16,), jnp.float32)

    def zstep(j, carry):
        pltpu.sync_copy(zb2, agg_sh.at[pl.ds(s * CNT_PER_TILE + j * 16, 16)])
        pltpu.sync_copy(zb1, cnt_sh.at[pl.ds(s * CNT_PER_TILE + j * 16, 16)])
        return carry

    lax.fori_loop(0, CNT_PER_TILE // 16, zstep, 0)

    @pl.when(s == NS - 1)
    def _():
        pltpu.sync_copy(zb2, agg_sh.at[pl.ds(NS * CNT_PER_TILE, CNT_TAIL)])
        pltpu.sync_copy(zb1, cnt_sh.at[pl.ds(NS * CNT_PER_TILE, CNT_TAIL)])

    plsc.subcore_barrier()

    row0 = wid * K2_ROWS_PER_W

    def step(i, carry):
        base = row0 + i * K2_INNER
        pltpu.sync_copy(src2d.at[pl.ds(base, K2_INNER)], idx_sv)
        pltpu.sync_copy(dst2d.at[pl.ds(base, K2_INNER)], idx_dv)
        cps = [
            pltpu.async_copy(
                h_hbm.at[idx_sv.at[j]], rows_v.at[pl.ds(j * K2_CH, K2_CH)], sem
            )
            for j in range(K2_INNER)
        ]
        for cp in cps:
            cp.wait()
        for j in range(K2_INNER):
            pltpu.sync_copy(
                rows_v.at[pl.ds(j * K2_CH, K2_CH)], agg_sh.at[idx_dv.at[j]], add=True
            )
            pltpu.sync_copy(ones_v.at[pl.ds(0, K2_CH)], cnt_sh.at[idx_dv.at[j]], add=True)
        return carry

    lax.fori_loop(0, K2_OUTER, step, 0)
    plsc.subcore_barrier()

    chunk = 104
    for part in range(CNT_PER_TILE // chunk):
        slp = pl.ds(s * CNT_PER_TILE + part * chunk, chunk)
        pltpu.sync_copy(agg_sh.at[slp], rows_v.at[pl.ds(0, chunk)])
        pltpu.sync_copy(rows_v.at[pl.ds(0, chunk)], agg_out.at[c, slp])
    pltpu.sync_copy(cnt_sh.at[pl.ds(s * CNT_PER_TILE, CNT_PER_TILE)], cz_v)
    pltpu.sync_copy(
        cz_v, cnt_out.at[pl.ds(c * N_NODES_C + s * CNT_PER_TILE, CNT_PER_TILE)]
    )

    @pl.when(s == NS - 1)
    def _():
        slt = pl.ds(NS * CNT_PER_TILE, CNT_TAIL)
        pltpu.sync_copy(agg_sh.at[slt], rows_v.at[pl.ds(0, CNT_TAIL)])
        pltpu.sync_copy(rows_v.at[pl.ds(0, CNT_TAIL)], agg_out.at[c, slt])
        pltpu.sync_copy(cnt_sh.at[slt], cz_v.at[pl.ds(0, CNT_TAIL)])
        pltpu.sync_copy(
            cz_v.at[pl.ds(0, CNT_TAIL)],
            cnt_out.at[pl.ds(c * N_NODES_C + NS * CNT_PER_TILE, CNT_TAIL)],
        )


def _gather_sc_body(eidx2d, pp_hbm, g_out, idx_v, g_v, sem):
    c = lax.axis_index("c")
    s = lax.axis_index("s")
    wid = s * NC + c
    row0 = wid * K4_OUTER * K4_INNER

    def step(i, carry):
        base = row0 + i * K4_INNER
        pltpu.sync_copy(eidx2d.at[pl.ds(base, K4_INNER)], idx_v)
        cps = [
            pltpu.async_copy(
                pp_hbm.at[idx_v.at[j]], g_v.at[pl.ds(j * K4_CH, K4_CH)], sem
            )
            for j in range(K4_INNER)
        ]
        for cp in cps:
            cp.wait()
        pltpu.sync_copy(g_v, g_out.at[pl.ds(base * K4_CH, K4_INNER * K4_CH)])
        return carry

    lax.fori_loop(0, K4_OUTER, step, 0)


def kernel(x, edge_index, edge_attr, W_node, b_node, W_edge, b_edge,
           W_l, b_l, W_r, W_cls, b_cls):
    n, d_in = x.shape
    e = edge_attr.shape[0]
    hid = W_node.shape[1]
    out_d = W_cls.shape[1]

    h = pl.pallas_call(
        _enc_body,
        out_shape=jax.ShapeDtypeStruct((n, hid), jnp.float32),
    )(x, W_node, b_node.reshape(1, hid))

    src2d = edge_index[0].reshape(e // K2_CH, K2_CH)
    dst2d = edge_index[1].reshape(e // K2_CH, K2_CH)

    agg2, cnt2 = pl.kernel(
        _agg_sc_body,
        out_type=(
            jax.ShapeDtypeStruct((NC, n, hid), jnp.float32),
            jax.ShapeDtypeStruct((NC * n,), jnp.float32),
        ),
        mesh=plsc.VectorSubcoreMesh(core_axis_name="c", subcore_axis_name="s", num_cores=NC, num_subcores=NS),
        scratch_types=[
            pltpu.VMEM((K2_INNER, K2_CH), jnp.int32),
            pltpu.VMEM((K2_INNER, K2_CH), jnp.int32),
            pltpu.VMEM((K2_INNER * K2_CH, hid), jnp.float32),
            pltpu.VMEM((64,), jnp.float32),
            pltpu.VMEM((16, hid), jnp.float32),
            pltpu.VMEM((16,), jnp.float32),
            pltpu.VMEM((CNT_PER_TILE,), jnp.float32),
            pltpu.VMEM_SHARED((n, hid), jnp.float32),
            pltpu.VMEM_SHARED((n,), jnp.float32),
            pltpu.SemaphoreType.DMA,
        ],
    )(src2d, dst2d, h)
    cnt2 = cnt2.reshape(NC, n)

    PPW = 16
    W_cls_p = jnp.pad(W_cls, ((0, 0), (0, PPW - out_d)))
    pp = pl.pallas_call(
        _sage_body,
        out_shape=jax.ShapeDtypeStruct((n, PPW), jnp.float32),
    )(agg2, cnt2, h, W_l, b_l.reshape(1, hid), W_r, W_cls_p)

    eidx2d = edge_index.reshape(2 * e // K4_CH, K4_CH)
    g = pl.kernel(
        _gather_sc_body,
        out_type=jax.ShapeDtypeStruct((2 * e, PPW), jnp.float32),
        mesh=plsc.VectorSubcoreMesh(core_axis_name="c", subcore_axis_name="s", num_cores=NC, num_subcores=NS),
        compiler_params=pltpu.CompilerParams(use_tc_tiling_on_sc=False),
        scratch_types=[
            pltpu.VMEM((K4_INNER, K4_CH), jnp.int32),
            pltpu.VMEM((K4_INNER * K4_CH, PPW), jnp.float32),
            pltpu.SemaphoreType.DMA,
        ],
    )(eidx2d, pp)

    blk = 5000
    out = pl.pallas_call(
        _edge_body,
        grid=(e // blk,),
        in_specs=[
            pl.BlockSpec((blk, PPW), lambda i: (i, 0)),
            pl.BlockSpec((blk, PPW), lambda i: (i, 0)),
            pl.BlockSpec((blk, edge_attr.shape[1]), lambda i: (i, 0)),
            pl.BlockSpec((edge_attr.shape[1], hid), lambda i: (0, 0)),
            pl.BlockSpec((1, hid), lambda i: (0, 0)),
            pl.BlockSpec((hid, out_d), lambda i: (0, 0)),
            pl.BlockSpec((1, out_d), lambda i: (0, 0)),
        ],
        out_specs=pl.BlockSpec((blk, out_d), lambda i: (i, 0)),
        out_shape=jax.ShapeDtypeStruct((e, out_d), jnp.float32),
    )(g[:e], g[e:], edge_attr, W_edge, b_edge.reshape(1, hid),
      W_cls, b_cls.reshape(1, out_d))

    return out


# speedup vs baseline: 1.5259x; 1.3658x over previous
import functools

import jax
import jax.numpy as jnp
from jax import lax
from jax.experimental import pallas as pl
from jax.experimental.pallas import tpu as pltpu
from jax.experimental.pallas import tpu_sc as plsc

N_NODES_C = 10000
N_EDGES_C = 320000
HID_C = 128

NC = 2
NS = 16
NW = NC * NS

K2_CH = 25
K4_CH = 50
K2_ROWS_PER_W = (N_EDGES_C // K2_CH) // NW
K2_INNER = 8
K2_OUTER = K2_ROWS_PER_W // K2_INNER
NODES_PER_TILE = N_NODES_C // NS
CNT_PER_TILE = 624
CNT_TAIL = N_NODES_C - NS * CNT_PER_TILE

K4_INNER = 8
K4_OUTER = (N_EDGES_C // K4_CH) // NW // K4_INNER


def _enc_body(x_ref, w_ref, b_ref, o_ref):
    o_ref[...] = jax.nn.relu(
        jnp.dot(x_ref[...], w_ref[...], preferred_element_type=jnp.float32)
        + b_ref[...]
    )


def _sage_body(agg_ref, cnt_ref, h_ref, wl_ref, bl_ref, wr_ref, wcls_ref, o_ref):
    agg = agg_ref[0] + agg_ref[1]
    cnt = cnt_ref[0] + cnt_ref[1]
    mean = agg / jnp.maximum(cnt, 1.0)[:, None]
    h2 = (
        jnp.dot(mean, wl_ref[...], preferred_element_type=jnp.float32)
        + bl_ref[...]
        + jnp.dot(h_ref[...], wr_ref[...], preferred_element_type=jnp.float32)
    )
    o_ref[...] = jnp.dot(h2, wcls_ref[...], preferred_element_type=jnp.float32) * (1.0 / 3.0)


def _edge_body(g_ref, ea_ref, we_ref, be_ref, wcls_ref, bcls_ref, o_ref):
    ea = jax.nn.relu(
        jnp.dot(ea_ref[...], we_ref[...], preferred_element_type=jnp.float32)
        + be_ref[...]
    )
    out_d = o_ref.shape[1]
    o_ref[...] = (
        g_ref[...][:, :out_d]
        + jnp.dot(ea, wcls_ref[...], preferred_element_type=jnp.float32) * (1.0 / 3.0)
        + bcls_ref[...]
    )


def _agg_sc_body(src2d, dst2d, h_hbm, agg_out, cnt_out,
                 idx_sv, idx_dv, rows_v, ones_v, zb2, zb1, cz_v,
                 agg_sh, cnt_sh, sem):
    c = lax.axis_index("c")
    s = lax.axis_index("s")
    wid = s * NC + c

    zv = jnp.zeros((16,), jnp.float32)
    for r in range(16):
        for c8 in range(8):
            zb2[r, pl.ds(c8 * 16, 16)] = zv
    zb1[pl.ds(0, 16)] = zv
    for i in range(4):
        ones_v[pl.ds(i * 16, 16)] = jnp.ones((16,), jnp.float32)

    def zstep(j, carry):
        pltpu.sync_copy(zb2, agg_sh.at[pl.ds(s * CNT_PER_TILE + j * 16, 16)])
        pltpu.sync_copy(zb1, cnt_sh.at[pl.ds(s * CNT_PER_TILE + j * 16, 16)])
        return carry

    lax.fori_loop(0, CNT_PER_TILE // 16, zstep, 0)

    @pl.when(s == NS - 1)
    def _():
        pltpu.sync_copy(zb2, agg_sh.at[pl.ds(NS * CNT_PER_TILE, CNT_TAIL)])
        pltpu.sync_copy(zb1, cnt_sh.at[pl.ds(NS * CNT_PER_TILE, CNT_TAIL)])

    plsc.subcore_barrier()

    row0 = wid * K2_ROWS_PER_W

    def step(i, carry):
        base = row0 + i * K2_INNER
        pltpu.sync_copy(src2d.at[pl.ds(base, K2_INNER)], idx_sv)
        pltpu.sync_copy(dst2d.at[pl.ds(base, K2_INNER)], idx_dv)
        cps = [
            pltpu.async_copy(
                h_hbm.at[idx_sv.at[j]], rows_v.at[pl.ds(j * K2_CH, K2_CH)], sem
            )
            for j in range(K2_INNER)
        ]
        for cp in cps:
            cp.wait()
        for j in range(K2_INNER):
            pltpu.sync_copy(
                rows_v.at[pl.ds(j * K2_CH, K2_CH)], agg_sh.at[idx_dv.at[j]], add=True
            )
            pltpu.sync_copy(ones_v.at[pl.ds(0, K2_CH)], cnt_sh.at[idx_dv.at[j]], add=True)
        return carry

    lax.fori_loop(0, K2_OUTER, step, 0)
    plsc.subcore_barrier()

    chunk = 104
    for part in range(CNT_PER_TILE // chunk):
        slp = pl.ds(s * CNT_PER_TILE + part * chunk, chunk)
        pltpu.sync_copy(agg_sh.at[slp], rows_v.at[pl.ds(0, chunk)])
        pltpu.sync_copy(rows_v.at[pl.ds(0, chunk)], agg_out.at[c, slp])
    pltpu.sync_copy(cnt_sh.at[pl.ds(s * CNT_PER_TILE, CNT_PER_TILE)], cz_v)
    pltpu.sync_copy(
        cz_v, cnt_out.at[pl.ds(c * N_NODES_C + s * CNT_PER_TILE, CNT_PER_TILE)]
    )

    @pl.when(s == NS - 1)
    def _():
        slt = pl.ds(NS * CNT_PER_TILE, CNT_TAIL)
        pltpu.sync_copy(agg_sh.at[slt], rows_v.at[pl.ds(0, CNT_TAIL)])
        pltpu.sync_copy(rows_v.at[pl.ds(0, CNT_TAIL)], agg_out.at[c, slt])
        pltpu.sync_copy(cnt_sh.at[slt], cz_v.at[pl.ds(0, CNT_TAIL)])
        pltpu.sync_copy(
            cz_v.at[pl.ds(0, CNT_TAIL)],
            cnt_out.at[pl.ds(c * N_NODES_C + NS * CNT_PER_TILE, CNT_TAIL)],
        )


def _gather_sc_body(src2d, dst2d, pp_hbm, g_out,
                    idx_sv, idx_dv, g_v, g_w, sem):
    c = lax.axis_index("c")
    s = lax.axis_index("s")
    wid = s * NC + c
    row0 = wid * K4_OUTER * K4_INNER

    def step(i, carry):
        base = row0 + i * K4_INNER
        pltpu.sync_copy(src2d.at[pl.ds(base, K4_INNER)], idx_sv)
        pltpu.sync_copy(dst2d.at[pl.ds(base, K4_INNER)], idx_dv)
        cps = [
            pltpu.async_copy(
                pp_hbm.at[idx_sv.at[j]], g_v.at[pl.ds(j * K4_CH, K4_CH)], sem
            )
            for j in range(K4_INNER)
        ] + [
            pltpu.async_copy(
                pp_hbm.at[idx_dv.at[j]], g_w.at[pl.ds(j * K4_CH, K4_CH)], sem
            )
            for j in range(K4_INNER)
        ]
        for cp in cps:
            cp.wait()

        def vadd(r, carry2):
            g_v[r] = g_v[r] + g_w[r]
            return carry2

        lax.fori_loop(0, K4_INNER * K4_CH, vadd, 0)
        pltpu.sync_copy(g_v, g_out.at[pl.ds(base * K4_CH, K4_INNER * K4_CH)])
        return carry

    lax.fori_loop(0, K4_OUTER, step, 0)


def kernel(x, edge_index, edge_attr, W_node, b_node, W_edge, b_edge,
           W_l, b_l, W_r, W_cls, b_cls):
    n, d_in = x.shape
    e = edge_attr.shape[0]
    hid = W_node.shape[1]
    out_d = W_cls.shape[1]

    h = pl.pallas_call(
        _enc_body,
        out_shape=jax.ShapeDtypeStruct((n, hid), jnp.float32),
    )(x, W_node, b_node.reshape(1, hid))

    src2d = edge_index[0].reshape(e // K2_CH, K2_CH)
    dst2d = edge_index[1].reshape(e // K2_CH, K2_CH)

    agg2, cnt2 = pl.kernel(
        _agg_sc_body,
        out_type=(
            jax.ShapeDtypeStruct((NC, n, hid), jnp.float32),
            jax.ShapeDtypeStruct((NC * n,), jnp.float32),
        ),
        mesh=plsc.VectorSubcoreMesh(core_axis_name="c", subcore_axis_name="s", num_cores=NC, num_subcores=NS),
        scratch_types=[
            pltpu.VMEM((K2_INNER, K2_CH), jnp.int32),
            pltpu.VMEM((K2_INNER, K2_CH), jnp.int32),
            pltpu.VMEM((K2_INNER * K2_CH, hid), jnp.float32),
            pltpu.VMEM((64,), jnp.float32),
            pltpu.VMEM((16, hid), jnp.float32),
            pltpu.VMEM((16,), jnp.float32),
            pltpu.VMEM((CNT_PER_TILE,), jnp.float32),
            pltpu.VMEM_SHARED((n, hid), jnp.float32),
            pltpu.VMEM_SHARED((n,), jnp.float32),
            pltpu.SemaphoreType.DMA,
        ],
    )(src2d, dst2d, h)
    cnt2 = cnt2.reshape(NC, n)

    PPW = 16
    W_cls_p = jnp.pad(W_cls, ((0, 0), (0, PPW - out_d)))
    pp = pl.pallas_call(
        _sage_body,
        out_shape=jax.ShapeDtypeStruct((n, PPW), jnp.float32),
    )(agg2, cnt2, h, W_l, b_l.reshape(1, hid), W_r, W_cls_p)

    src2d4 = edge_index[0].reshape(e // K4_CH, K4_CH)
    dst2d4 = edge_index[1].reshape(e // K4_CH, K4_CH)
    g = pl.kernel(
        _gather_sc_body,
        out_type=jax.ShapeDtypeStruct((e, PPW), jnp.float32),
        mesh=plsc.VectorSubcoreMesh(core_axis_name="c", subcore_axis_name="s", num_cores=NC, num_subcores=NS),
        compiler_params=pltpu.CompilerParams(use_tc_tiling_on_sc=False),
        scratch_types=[
            pltpu.VMEM((K4_INNER, K4_CH), jnp.int32),
            pltpu.VMEM((K4_INNER, K4_CH), jnp.int32),
            pltpu.VMEM((K4_INNER * K4_CH, PPW), jnp.float32),
            pltpu.VMEM((K4_INNER * K4_CH, PPW), jnp.float32),
            pltpu.SemaphoreType.DMA,
        ],
    )(src2d4, dst2d4, pp)

    blk = 5000
    out = pl.pallas_call(
        _edge_body,
        grid=(e // blk,),
        in_specs=[
            pl.BlockSpec((blk, PPW), lambda i: (i, 0)),
            pl.BlockSpec((blk, edge_attr.shape[1]), lambda i: (i, 0)),
            pl.BlockSpec((edge_attr.shape[1], hid), lambda i: (0, 0)),
            pl.BlockSpec((1, hid), lambda i: (0, 0)),
            pl.BlockSpec((hid, out_d), lambda i: (0, 0)),
            pl.BlockSpec((1, out_d), lambda i: (0, 0)),
        ],
        out_specs=pl.BlockSpec((blk, out_d), lambda i: (i, 0)),
        out_shape=jax.ShapeDtypeStruct((e, out_d), jnp.float32),
    )(g, edge_attr, W_edge, b_edge.reshape(1, hid),
      W_cls, b_cls.reshape(1, out_d))

    return out


# speedup vs baseline: 1.6347x; 1.0713x over previous
import functools

import jax
import jax.numpy as jnp
from jax import lax
from jax.experimental import pallas as pl
from jax.experimental.pallas import tpu as pltpu
from jax.experimental.pallas import tpu_sc as plsc

N_NODES_C = 10000
N_EDGES_C = 320000
HID_C = 128

NC = 2
NS = 16
NW = NC * NS

K2_CH = 50
K4_CH = 50
K2_ROWS_PER_W = (N_EDGES_C // K2_CH) // NW
K2_INNER = 4
K2_OUTER = K2_ROWS_PER_W // K2_INNER
NODES_PER_TILE = N_NODES_C // NS
CNT_PER_TILE = 624
CNT_TAIL = N_NODES_C - NS * CNT_PER_TILE

K4_INNER = 8
K4_OUTER = (N_EDGES_C // K4_CH) // NW // K4_INNER


def _enc_body(x_ref, w_ref, b_ref, o_ref):
    o_ref[...] = jax.nn.relu(
        jnp.dot(x_ref[...], w_ref[...], preferred_element_type=jnp.float32)
        + b_ref[...]
    )


def _sage_body(agg_ref, cnt_ref, h_ref, wl_ref, bl_ref, wr_ref, wcls_ref, o_ref):
    agg = agg_ref[0] + agg_ref[1]
    cnt = cnt_ref[0] + cnt_ref[1]
    mean = agg / jnp.maximum(cnt, 1.0)[:, None]
    h2 = (
        jnp.dot(mean, wl_ref[...], preferred_element_type=jnp.float32)
        + bl_ref[...]
        + jnp.dot(h_ref[...], wr_ref[...], preferred_element_type=jnp.float32)
    )
    o_ref[...] = jnp.dot(h2, wcls_ref[...], preferred_element_type=jnp.float32) * (1.0 / 3.0)


def _edge_body(g_ref, ea_ref, we_ref, be_ref, wcls_ref, bcls_ref, o_ref):
    ea = jax.nn.relu(
        jnp.dot(ea_ref[...], we_ref[...], preferred_element_type=jnp.float32)
        + be_ref[...]
    )
    out_d = o_ref.shape[1]
    o_ref[...] = (
        g_ref[...][:, :out_d]
        + jnp.dot(ea, wcls_ref[...], preferred_element_type=jnp.float32) * (1.0 / 3.0)
        + bcls_ref[...]
    )


def _agg_sc_body(src2d, dst2d, h_hbm, agg_out, cnt_out,
                 idx_sv, idx_dv, rows_v, ones_v, zb2, zb1, cz_v,
                 agg_sh, cnt_sh, sem):
    c = lax.axis_index("c")
    s = lax.axis_index("s")
    wid = s * NC + c

    zv = jnp.zeros((16,), jnp.float32)
    for r in range(16):
        for c8 in range(8):
            zb2[r, pl.ds(c8 * 16, 16)] = zv
    zb1[pl.ds(0, 16)] = zv
    for i in range(4):
        ones_v[pl.ds(i * 16, 16)] = jnp.ones((16,), jnp.float32)

    def zstep(j, carry):
        pltpu.sync_copy(zb2, agg_sh.at[pl.ds(s * CNT_PER_TILE + j * 16, 16)])
        pltpu.sync_copy(zb1, cnt_sh.at[pl.ds(s * CNT_PER_TILE + j * 16, 16)])
        return carry

    lax.fori_loop(0, CNT_PER_TILE // 16, zstep, 0)

    @pl.when(s == NS - 1)
    def _():
        pltpu.sync_copy(zb2, agg_sh.at[pl.ds(NS * CNT_PER_TILE, CNT_TAIL)])
        pltpu.sync_copy(zb1, cnt_sh.at[pl.ds(NS * CNT_PER_TILE, CNT_TAIL)])

    plsc.subcore_barrier()

    row0 = wid * K2_ROWS_PER_W

    def step(i, carry):
        base = row0 + i * K2_INNER
        pltpu.sync_copy(src2d.at[pl.ds(base, K2_INNER)], idx_sv)
        pltpu.sync_copy(dst2d.at[pl.ds(base, K2_INNER)], idx_dv)
        cps = [
            pltpu.async_copy(
                h_hbm.at[idx_sv.at[j]], rows_v.at[pl.ds(j * K2_CH, K2_CH)], sem
            )
            for j in range(K2_INNER)
        ]
        for cp in cps:
            cp.wait()
        for j in range(K2_INNER):
            pltpu.sync_copy(
                rows_v.at[pl.ds(j * K2_CH, K2_CH)], agg_sh.at[idx_dv.at[j]], add=True
            )
            pltpu.sync_copy(ones_v.at[pl.ds(0, K2_CH)], cnt_sh.at[idx_dv.at[j]], add=True)
        return carry

    lax.fori_loop(0, K2_OUTER, step, 0)
    plsc.subcore_barrier()

    chunk = 104
    for part in range(CNT_PER_TILE // chunk):
        slp = pl.ds(s * CNT_PER_TILE + part * chunk, chunk)
        pltpu.sync_copy(agg_sh.at[slp], rows_v.at[pl.ds(0, chunk)])
        pltpu.sync_copy(rows_v.at[pl.ds(0, chunk)], agg_out.at[c, slp])
    pltpu.sync_copy(cnt_sh.at[pl.ds(s * CNT_PER_TILE, CNT_PER_TILE)], cz_v)
    pltpu.sync_copy(
        cz_v, cnt_out.at[pl.ds(c * N_NODES_C + s * CNT_PER_TILE, CNT_PER_TILE)]
    )

    @pl.when(s == NS - 1)
    def _():
        slt = pl.ds(NS * CNT_PER_TILE, CNT_TAIL)
        pltpu.sync_copy(agg_sh.at[slt], rows_v.at[pl.ds(0, CNT_TAIL)])
        pltpu.sync_copy(rows_v.at[pl.ds(0, CNT_TAIL)], agg_out.at[c, slt])
        pltpu.sync_copy(cnt_sh.at[slt], cz_v.at[pl.ds(0, CNT_TAIL)])
        pltpu.sync_copy(
            cz_v.at[pl.ds(0, CNT_TAIL)],
            cnt_out.at[pl.ds(c * N_NODES_C + NS * CNT_PER_TILE, CNT_TAIL)],
        )


def _gather_sc_body(src2d, dst2d, pp_hbm, g_out,
                    idx_sv, idx_dv, g_v, g_w, sem):
    c = lax.axis_index("c")
    s = lax.axis_index("s")
    wid = s * NC + c
    row0 = wid * K4_OUTER * K4_INNER

    def step(i, carry):
        base = row0 + i * K4_INNER
        pltpu.sync_copy(src2d.at[pl.ds(base, K4_INNER)], idx_sv)
        pltpu.sync_copy(dst2d.at[pl.ds(base, K4_INNER)], idx_dv)
        cps = [
            pltpu.async_copy(
                pp_hbm.at[idx_sv.at[j]], g_v.at[pl.ds(j * K4_CH, K4_CH)], sem
            )
            for j in range(K4_INNER)
        ] + [
            pltpu.async_copy(
                pp_hbm.at[idx_dv.at[j]], g_w.at[pl.ds(j * K4_CH, K4_CH)], sem
            )
            for j in range(K4_INNER)
        ]
        for cp in cps:
            cp.wait()

        def vadd(r, carry2):
            g_v[r] = g_v[r] + g_w[r]
            return carry2

        lax.fori_loop(0, K4_INNER * K4_CH, vadd, 0)
        pltpu.sync_copy(g_v, g_out.at[pl.ds(base * K4_CH, K4_INNER * K4_CH)])
        return carry

    lax.fori_loop(0, K4_OUTER, step, 0)


def kernel(x, edge_index, edge_attr, W_node, b_node, W_edge, b_edge,
           W_l, b_l, W_r, W_cls, b_cls):
    n, d_in = x.shape
    e = edge_attr.shape[0]
    hid = W_node.shape[1]
    out_d = W_cls.shape[1]

    h = pl.pallas_call(
        _enc_body,
        out_shape=jax.ShapeDtypeStruct((n, hid), jnp.float32),
    )(x, W_node, b_node.reshape(1, hid))

    src2d = edge_index[0].reshape(e // K2_CH, K2_CH)
    dst2d = edge_index[1].reshape(e // K2_CH, K2_CH)

    agg2, cnt2 = pl.kernel(
        _agg_sc_body,
        out_type=(
            jax.ShapeDtypeStruct((NC, n, hid), jnp.float32),
            jax.ShapeDtypeStruct((NC * n,), jnp.float32),
        ),
        mesh=plsc.VectorSubcoreMesh(core_axis_name="c", subcore_axis_name="s", num_cores=NC, num_subcores=NS),
        scratch_types=[
            pltpu.VMEM((K2_INNER, K2_CH), jnp.int32),
            pltpu.VMEM((K2_INNER, K2_CH), jnp.int32),
            pltpu.VMEM((K2_INNER * K2_CH, hid), jnp.float32),
            pltpu.VMEM((64,), jnp.float32),
            pltpu.VMEM((16, hid), jnp.float32),
            pltpu.VMEM((16,), jnp.float32),
            pltpu.VMEM((CNT_PER_TILE,), jnp.float32),
            pltpu.VMEM_SHARED((n, hid), jnp.float32),
            pltpu.VMEM_SHARED((n,), jnp.float32),
            pltpu.SemaphoreType.DMA,
        ],
    )(src2d, dst2d, h)
    cnt2 = cnt2.reshape(NC, n)

    PPW = 16
    W_cls_p = jnp.pad(W_cls, ((0, 0), (0, PPW - out_d)))
    pp = pl.pallas_call(
        _sage_body,
        out_shape=jax.ShapeDtypeStruct((n, PPW), jnp.float32),
    )(agg2, cnt2, h, W_l, b_l.reshape(1, hid), W_r, W_cls_p)

    src2d4 = edge_index[0].reshape(e // K4_CH, K4_CH)
    dst2d4 = edge_index[1].reshape(e // K4_CH, K4_CH)
    g = pl.kernel(
        _gather_sc_body,
        out_type=jax.ShapeDtypeStruct((e, PPW), jnp.float32),
        mesh=plsc.VectorSubcoreMesh(core_axis_name="c", subcore_axis_name="s", num_cores=NC, num_subcores=NS),
        compiler_params=pltpu.CompilerParams(use_tc_tiling_on_sc=False),
        scratch_types=[
            pltpu.VMEM((K4_INNER, K4_CH), jnp.int32),
            pltpu.VMEM((K4_INNER, K4_CH), jnp.int32),
            pltpu.VMEM((K4_INNER * K4_CH, PPW), jnp.float32),
            pltpu.VMEM((K4_INNER * K4_CH, PPW), jnp.float32),
            pltpu.SemaphoreType.DMA,
        ],
    )(src2d4, dst2d4, pp)

    blk = 10000
    out = pl.pallas_call(
        _edge_body,
        grid=(e // blk,),
        in_specs=[
            pl.BlockSpec((blk, PPW), lambda i: (i, 0)),
            pl.BlockSpec((blk, edge_attr.shape[1]), lambda i: (i, 0)),
            pl.BlockSpec((edge_attr.shape[1], hid), lambda i: (0, 0)),
            pl.BlockSpec((1, hid), lambda i: (0, 0)),
            pl.BlockSpec((hid, out_d), lambda i: (0, 0)),
            pl.BlockSpec((1, out_d), lambda i: (0, 0)),
        ],
        out_specs=pl.BlockSpec((blk, out_d), lambda i: (i, 0)),
        out_shape=jax.ShapeDtypeStruct((e, out_d), jnp.float32),
    )(g, edge_attr, W_edge, b_edge.reshape(1, hid),
      W_cls, b_cls.reshape(1, out_d))

    return out


# speedup vs baseline: 1.6843x; 1.0303x over previous
import functools

import jax
import jax.numpy as jnp
from jax import lax
from jax.experimental import pallas as pl
from jax.experimental.pallas import tpu as pltpu
from jax.experimental.pallas import tpu_sc as plsc

N_NODES_C = 10000
N_EDGES_C = 320000
HID_C = 128

NC = 2
NS = 16
NW = NC * NS

K2_CH = 100
K4_CH = 100
K2_ROWS_PER_W = (N_EDGES_C // K2_CH) // NW
K2_INNER = 2
K2_OUTER = K2_ROWS_PER_W // K2_INNER
NODES_PER_TILE = N_NODES_C // NS
CNT_PER_TILE = 624
CNT_TAIL = N_NODES_C - NS * CNT_PER_TILE

K4_INNER = 4
K4_OUTER = (N_EDGES_C // K4_CH) // NW // K4_INNER


def _enc_body(x_ref, w_ref, b_ref, o_ref):
    o_ref[...] = jax.nn.relu(
        jnp.dot(x_ref[...], w_ref[...], preferred_element_type=jnp.float32)
        + b_ref[...]
    )


def _sage_body(agg_ref, cnt_ref, h_ref, wl_ref, bl_ref, wr_ref, wcls_ref, o_ref):
    agg = agg_ref[0] + agg_ref[1]
    cnt = cnt_ref[0] + cnt_ref[1]
    mean = agg / jnp.maximum(cnt, 1.0)[:, None]
    h2 = (
        jnp.dot(mean, wl_ref[...], preferred_element_type=jnp.float32)
        + bl_ref[...]
        + jnp.dot(h_ref[...], wr_ref[...], preferred_element_type=jnp.float32)
    )
    o_ref[...] = jnp.dot(h2, wcls_ref[...], preferred_element_type=jnp.float32) * (1.0 / 3.0)


def _edge_body(g_ref, ea_ref, we_ref, be_ref, wcls_ref, bcls_ref, o_ref):
    ea = jax.nn.relu(
        jnp.dot(ea_ref[...], we_ref[...], preferred_element_type=jnp.float32)
        + be_ref[...]
    )
    out_d = o_ref.shape[1]
    o_ref[...] = (
        g_ref[...][:, :out_d]
        + jnp.dot(ea, wcls_ref[...], preferred_element_type=jnp.float32) * (1.0 / 3.0)
        + bcls_ref[...]
    )


def _agg_sc_body(src2d, dst2d, h_hbm, agg_out, cnt_out,
                 idx_sv, idx_dv, rows_v, ones_v, zb2, zb1, cz_v,
                 agg_sh, cnt_sh, sem):
    c = lax.axis_index("c")
    s = lax.axis_index("s")
    wid = s * NC + c

    zv = jnp.zeros((16,), jnp.float32)
    for r in range(16):
        for c8 in range(8):
            zb2[r, pl.ds(c8 * 16, 16)] = zv
    zb1[pl.ds(0, 16)] = zv
    for i in range(8):
        ones_v[pl.ds(i * 16, 16)] = jnp.ones((16,), jnp.float32)

    def zstep(j, carry):
        pltpu.sync_copy(zb2, agg_sh.at[pl.ds(s * CNT_PER_TILE + j * 16, 16)])
        pltpu.sync_copy(zb1, cnt_sh.at[pl.ds(s * CNT_PER_TILE + j * 16, 16)])
        return carry

    lax.fori_loop(0, CNT_PER_TILE // 16, zstep, 0)

    @pl.when(s == NS - 1)
    def _():
        pltpu.sync_copy(zb2, agg_sh.at[pl.ds(NS * CNT_PER_TILE, CNT_TAIL)])
        pltpu.sync_copy(zb1, cnt_sh.at[pl.ds(NS * CNT_PER_TILE, CNT_TAIL)])

    plsc.subcore_barrier()

    row0 = wid * K2_ROWS_PER_W

    def step(i, carry):
        base = row0 + i * K2_INNER
        pltpu.sync_copy(src2d.at[pl.ds(base, K2_INNER)], idx_sv)
        pltpu.sync_copy(dst2d.at[pl.ds(base, K2_INNER)], idx_dv)
        cps = [
            pltpu.async_copy(
                h_hbm.at[idx_sv.at[j]], rows_v.at[pl.ds(j * K2_CH, K2_CH)], sem
            )
            for j in range(K2_INNER)
        ]
        for cp in cps:
            cp.wait()
        for j in range(K2_INNER):
            pltpu.sync_copy(
                rows_v.at[pl.ds(j * K2_CH, K2_CH)], agg_sh.at[idx_dv.at[j]], add=True
            )
            pltpu.sync_copy(ones_v.at[pl.ds(0, K2_CH)], cnt_sh.at[idx_dv.at[j]], add=True)
        return carry

    lax.fori_loop(0, K2_OUTER, step, 0)
    plsc.subcore_barrier()

    chunk = 104
    for part in range(CNT_PER_TILE // chunk):
        slp = pl.ds(s * CNT_PER_TILE + part * chunk, chunk)
        pltpu.sync_copy(agg_sh.at[slp], rows_v.at[pl.ds(0, chunk)])
        pltpu.sync_copy(rows_v.at[pl.ds(0, chunk)], agg_out.at[c, slp])
    pltpu.sync_copy(cnt_sh.at[pl.ds(s * CNT_PER_TILE, CNT_PER_TILE)], cz_v)
    pltpu.sync_copy(
        cz_v, cnt_out.at[pl.ds(c * N_NODES_C + s * CNT_PER_TILE, CNT_PER_TILE)]
    )

    @pl.when(s == NS - 1)
    def _():
        slt = pl.ds(NS * CNT_PER_TILE, CNT_TAIL)
        pltpu.sync_copy(agg_sh.at[slt], rows_v.at[pl.ds(0, CNT_TAIL)])
        pltpu.sync_copy(rows_v.at[pl.ds(0, CNT_TAIL)], agg_out.at[c, slt])
        pltpu.sync_copy(cnt_sh.at[slt], cz_v.at[pl.ds(0, CNT_TAIL)])
        pltpu.sync_copy(
            cz_v.at[pl.ds(0, CNT_TAIL)],
            cnt_out.at[pl.ds(c * N_NODES_C + NS * CNT_PER_TILE, CNT_TAIL)],
        )


def _gather_sc_body(src2d, dst2d, pp_hbm, g_out,
                    idx_sv, idx_dv, g_v, g_w, sem):
    c = lax.axis_index("c")
    s = lax.axis_index("s")
    wid = s * NC + c
    row0 = wid * K4_OUTER * K4_INNER

    def step(i, carry):
        base = row0 + i * K4_INNER
        pltpu.sync_copy(src2d.at[pl.ds(base, K4_INNER)], idx_sv)
        pltpu.sync_copy(dst2d.at[pl.ds(base, K4_INNER)], idx_dv)
        cps = [
            pltpu.async_copy(
                pp_hbm.at[idx_sv.at[j]], g_v.at[pl.ds(j * K4_CH, K4_CH)], sem
            )
            for j in range(K4_INNER)
        ] + [
            pltpu.async_copy(
                pp_hbm.at[idx_dv.at[j]], g_w.at[pl.ds(j * K4_CH, K4_CH)], sem
            )
            for j in range(K4_INNER)
        ]
        for cp in cps:
            cp.wait()

        def vadd(r, carry2):
            g_v[r] = g_v[r] + g_w[r]
            return carry2

        lax.fori_loop(0, K4_INNER * K4_CH, vadd, 0)
        pltpu.sync_copy(g_v, g_out.at[pl.ds(base * K4_CH, K4_INNER * K4_CH)])
        return carry

    lax.fori_loop(0, K4_OUTER, step, 0)


def kernel(x, edge_index, edge_attr, W_node, b_node, W_edge, b_edge,
           W_l, b_l, W_r, W_cls, b_cls):
    n, d_in = x.shape
    e = edge_attr.shape[0]
    hid = W_node.shape[1]
    out_d = W_cls.shape[1]

    h = pl.pallas_call(
        _enc_body,
        out_shape=jax.ShapeDtypeStruct((n, hid), jnp.float32),
    )(x, W_node, b_node.reshape(1, hid))

    src2d = edge_index[0].reshape(e // K2_CH, K2_CH)
    dst2d = edge_index[1].reshape(e // K2_CH, K2_CH)

    agg2, cnt2 = pl.kernel(
        _agg_sc_body,
        out_type=(
            jax.ShapeDtypeStruct((NC, n, hid), jnp.float32),
            jax.ShapeDtypeStruct((NC * n,), jnp.float32),
        ),
        mesh=plsc.VectorSubcoreMesh(core_axis_name="c", subcore_axis_name="s", num_cores=NC, num_subcores=NS),
        scratch_types=[
            pltpu.VMEM((K2_INNER, K2_CH), jnp.int32),
            pltpu.VMEM((K2_INNER, K2_CH), jnp.int32),
            pltpu.VMEM((K2_INNER * K2_CH, hid), jnp.float32),
            pltpu.VMEM((128,), jnp.float32),
            pltpu.VMEM((16, hid), jnp.float32),
            pltpu.VMEM((16,), jnp.float32),
            pltpu.VMEM((CNT_PER_TILE,), jnp.float32),
            pltpu.VMEM_SHARED((n, hid), jnp.float32),
            pltpu.VMEM_SHARED((n,), jnp.float32),
            pltpu.SemaphoreType.DMA,
        ],
    )(src2d, dst2d, h)
    cnt2 = cnt2.reshape(NC, n)

    PPW = 16
    W_cls_p = jnp.pad(W_cls, ((0, 0), (0, PPW - out_d)))
    pp = pl.pallas_call(
        _sage_body,
        out_shape=jax.ShapeDtypeStruct((n, PPW), jnp.float32),
    )(agg2, cnt2, h, W_l, b_l.reshape(1, hid), W_r, W_cls_p)

    src2d4 = edge_index[0].reshape(e // K4_CH, K4_CH)
    dst2d4 = edge_index[1].reshape(e // K4_CH, K4_CH)
    g = pl.kernel(
        _gather_sc_body,
        out_type=jax.ShapeDtypeStruct((e, PPW), jnp.float32),
        mesh=plsc.VectorSubcoreMesh(core_axis_name="c", subcore_axis_name="s", num_cores=NC, num_subcores=NS),
        compiler_params=pltpu.CompilerParams(use_tc_tiling_on_sc=False),
        scratch_types=[
            pltpu.VMEM((K4_INNER, K4_CH), jnp.int32),
            pltpu.VMEM((K4_INNER, K4_CH), jnp.int32),
            pltpu.VMEM((K4_INNER * K4_CH, PPW), jnp.float32),
            pltpu.VMEM((K4_INNER * K4_CH, PPW), jnp.float32),
            pltpu.SemaphoreType.DMA,
        ],
    )(src2d4, dst2d4, pp)

    blk = 10000
    out = pl.pallas_call(
        _edge_body,
        grid=(e // blk,),
        in_specs=[
            pl.BlockSpec((blk, PPW), lambda i: (i, 0)),
            pl.BlockSpec((blk, edge_attr.shape[1]), lambda i: (i, 0)),
            pl.BlockSpec((edge_attr.shape[1], hid), lambda i: (0, 0)),
            pl.BlockSpec((1, hid), lambda i: (0, 0)),
            pl.BlockSpec((hid, out_d), lambda i: (0, 0)),
            pl.BlockSpec((1, out_d), lambda i: (0, 0)),
        ],
        out_specs=pl.BlockSpec((blk, out_d), lambda i: (i, 0)),
        out_shape=jax.ShapeDtypeStruct((e, out_d), jnp.float32),
    )(g, edge_attr, W_edge, b_edge.reshape(1, hid),
      W_cls, b_cls.reshape(1, out_d))

    return out


# speedup vs baseline: 1.6982x; 1.0082x over previous
import functools

import jax
import jax.numpy as jnp
from jax import lax
from jax.experimental import pallas as pl
from jax.experimental.pallas import tpu as pltpu
from jax.experimental.pallas import tpu_sc as plsc

N_NODES_C = 10000
N_EDGES_C = 320000
HID_C = 128

NC = 2
NS = 16
NW = NC * NS

K2_CH = 200
K4_CH = 200
K2_ROWS_PER_W = (N_EDGES_C // K2_CH) // NW
K2_INNER = 1
K2_OUTER = K2_ROWS_PER_W // K2_INNER
NODES_PER_TILE = N_NODES_C // NS
CNT_PER_TILE = 624
CNT_TAIL = N_NODES_C - NS * CNT_PER_TILE

K4_INNER = 2
K4_OUTER = (N_EDGES_C // K4_CH) // NW // K4_INNER


def _enc_body(x_ref, w_ref, b_ref, o_ref):
    o_ref[...] = jax.nn.relu(
        jnp.dot(x_ref[...], w_ref[...], preferred_element_type=jnp.float32)
        + b_ref[...]
    )


def _sage_body(agg_ref, cnt_ref, h_ref, wl_ref, bl_ref, wr_ref, wcls_ref, o_ref):
    agg = agg_ref[0] + agg_ref[1]
    cnt = cnt_ref[0] + cnt_ref[1]
    mean = agg / jnp.maximum(cnt, 1.0)[:, None]
    h2 = (
        jnp.dot(mean, wl_ref[...], preferred_element_type=jnp.float32)
        + bl_ref[...]
        + jnp.dot(h_ref[...], wr_ref[...], preferred_element_type=jnp.float32)
    )
    o_ref[...] = jnp.dot(h2, wcls_ref[...], preferred_element_type=jnp.float32) * (1.0 / 3.0)


def _edge_body(g_ref, ea_ref, we_ref, be_ref, wcls_ref, bcls_ref, o_ref):
    ea = jax.nn.relu(
        jnp.dot(ea_ref[...], we_ref[...], preferred_element_type=jnp.float32)
        + be_ref[...]
    )
    out_d = o_ref.shape[1]
    o_ref[...] = (
        g_ref[...][:, :out_d]
        + jnp.dot(ea, wcls_ref[...], preferred_element_type=jnp.float32) * (1.0 / 3.0)
        + bcls_ref[...]
    )


def _agg_sc_body(src2d, dst2d, h_hbm, agg_out, cnt_out,
                 idx_sv, idx_dv, rows_v, ones_v, zb2, zb1, cz_v,
                 agg_sh, cnt_sh, sem):
    c = lax.axis_index("c")
    s = lax.axis_index("s")
    wid = s * NC + c

    zv = jnp.zeros((16,), jnp.float32)
    for r in range(16):
        for c8 in range(8):
            zb2[r, pl.ds(c8 * 16, 16)] = zv
    zb1[pl.ds(0, 16)] = zv
    for i in range(13):
        ones_v[pl.ds(i * 16, 16)] = jnp.ones((16,), jnp.float32)

    def zstep(j, carry):
        pltpu.sync_copy(zb2, agg_sh.at[pl.ds(s * CNT_PER_TILE + j * 16, 16)])
        pltpu.sync_copy(zb1, cnt_sh.at[pl.ds(s * CNT_PER_TILE + j * 16, 16)])
        return carry

    lax.fori_loop(0, CNT_PER_TILE // 16, zstep, 0)

    @pl.when(s == NS - 1)
    def _():
        pltpu.sync_copy(zb2, agg_sh.at[pl.ds(NS * CNT_PER_TILE, CNT_TAIL)])
        pltpu.sync_copy(zb1, cnt_sh.at[pl.ds(NS * CNT_PER_TILE, CNT_TAIL)])

    plsc.subcore_barrier()

    row0 = wid * K2_ROWS_PER_W

    def step(i, carry):
        base = row0 + i * K2_INNER
        pltpu.sync_copy(src2d.at[pl.ds(base, K2_INNER)], idx_sv)
        pltpu.sync_copy(dst2d.at[pl.ds(base, K2_INNER)], idx_dv)
        cps = [
            pltpu.async_copy(
                h_hbm.at[idx_sv.at[j]], rows_v.at[pl.ds(j * K2_CH, K2_CH)], sem
            )
            for j in range(K2_INNER)
        ]
        for cp in cps:
            cp.wait()
        for j in range(K2_INNER):
            pltpu.sync_copy(
                rows_v.at[pl.ds(j * K2_CH, K2_CH)], agg_sh.at[idx_dv.at[j]], add=True
            )
            pltpu.sync_copy(ones_v.at[pl.ds(0, K2_CH)], cnt_sh.at[idx_dv.at[j]], add=True)
        return carry

    lax.fori_loop(0, K2_OUTER, step, 0)
    plsc.subcore_barrier()

    chunk = 104
    for part in range(CNT_PER_TILE // chunk):
        slp = pl.ds(s * CNT_PER_TILE + part * chunk, chunk)
        pltpu.sync_copy(agg_sh.at[slp], rows_v.at[pl.ds(0, chunk)])
        pltpu.sync_copy(rows_v.at[pl.ds(0, chunk)], agg_out.at[c, slp])
    pltpu.sync_copy(cnt_sh.at[pl.ds(s * CNT_PER_TILE, CNT_PER_TILE)], cz_v)
    pltpu.sync_copy(
        cz_v, cnt_out.at[pl.ds(c * N_NODES_C + s * CNT_PER_TILE, CNT_PER_TILE)]
    )

    @pl.when(s == NS - 1)
    def _():
        slt = pl.ds(NS * CNT_PER_TILE, CNT_TAIL)
        pltpu.sync_copy(agg_sh.at[slt], rows_v.at[pl.ds(0, CNT_TAIL)])
        pltpu.sync_copy(rows_v.at[pl.ds(0, CNT_TAIL)], agg_out.at[c, slt])
        pltpu.sync_copy(cnt_sh.at[slt], cz_v.at[pl.ds(0, CNT_TAIL)])
        pltpu.sync_copy(
            cz_v.at[pl.ds(0, CNT_TAIL)],
            cnt_out.at[pl.ds(c * N_NODES_C + NS * CNT_PER_TILE, CNT_TAIL)],
        )


def _gather_sc_body(src2d, dst2d, pp_hbm, g_out,
                    idx_sv, idx_dv, g_v, g_w, sem):
    c = lax.axis_index("c")
    s = lax.axis_index("s")
    wid = s * NC + c
    row0 = wid * K4_OUTER * K4_INNER

    def step(i, carry):
        base = row0 + i * K4_INNER
        pltpu.sync_copy(src2d.at[pl.ds(base, K4_INNER)], idx_sv)
        pltpu.sync_copy(dst2d.at[pl.ds(base, K4_INNER)], idx_dv)
        cps = [
            pltpu.async_copy(
                pp_hbm.at[idx_sv.at[j]], g_v.at[pl.ds(j * K4_CH, K4_CH)], sem
            )
            for j in range(K4_INNER)
        ] + [
            pltpu.async_copy(
                pp_hbm.at[idx_dv.at[j]], g_w.at[pl.ds(j * K4_CH, K4_CH)], sem
            )
            for j in range(K4_INNER)
        ]
        for cp in cps:
            cp.wait()

        def vadd(r, carry2):
            g_v[r] = g_v[r] + g_w[r]
            return carry2

        lax.fori_loop(0, K4_INNER * K4_CH, vadd, 0)
        pltpu.sync_copy(g_v, g_out.at[pl.ds(base * K4_CH, K4_INNER * K4_CH)])
        return carry

    lax.fori_loop(0, K4_OUTER, step, 0)


def kernel(x, edge_index, edge_attr, W_node, b_node, W_edge, b_edge,
           W_l, b_l, W_r, W_cls, b_cls):
    n, d_in = x.shape
    e = edge_attr.shape[0]
    hid = W_node.shape[1]
    out_d = W_cls.shape[1]

    h = pl.pallas_call(
        _enc_body,
        out_shape=jax.ShapeDtypeStruct((n, hid), jnp.float32),
    )(x, W_node, b_node.reshape(1, hid))

    src2d = edge_index[0].reshape(e // K2_CH, K2_CH)
    dst2d = edge_index[1].reshape(e // K2_CH, K2_CH)

    agg2, cnt2 = pl.kernel(
        _agg_sc_body,
        out_type=(
            jax.ShapeDtypeStruct((NC, n, hid), jnp.float32),
            jax.ShapeDtypeStruct((NC * n,), jnp.float32),
        ),
        mesh=plsc.VectorSubcoreMesh(core_axis_name="c", subcore_axis_name="s", num_cores=NC, num_subcores=NS),
        scratch_types=[
            pltpu.VMEM((K2_INNER, K2_CH), jnp.int32),
            pltpu.VMEM((K2_INNER, K2_CH), jnp.int32),
            pltpu.VMEM((K2_INNER * K2_CH, hid), jnp.float32),
            pltpu.VMEM((208,), jnp.float32),
            pltpu.VMEM((16, hid), jnp.float32),
            pltpu.VMEM((16,), jnp.float32),
            pltpu.VMEM((CNT_PER_TILE,), jnp.float32),
            pltpu.VMEM_SHARED((n, hid), jnp.float32),
            pltpu.VMEM_SHARED((n,), jnp.float32),
            pltpu.SemaphoreType.DMA,
        ],
    )(src2d, dst2d, h)
    cnt2 = cnt2.reshape(NC, n)

    PPW = 16
    W_cls_p = jnp.pad(W_cls, ((0, 0), (0, PPW - out_d)))
    pp = pl.pallas_call(
        _sage_body,
        out_shape=jax.ShapeDtypeStruct((n, PPW), jnp.float32),
    )(agg2, cnt2, h, W_l, b_l.reshape(1, hid), W_r, W_cls_p)

    src2d4 = edge_index[0].reshape(e // K4_CH, K4_CH)
    dst2d4 = edge_index[1].reshape(e // K4_CH, K4_CH)
    g = pl.kernel(
        _gather_sc_body,
        out_type=jax.ShapeDtypeStruct((e, PPW), jnp.float32),
        mesh=plsc.VectorSubcoreMesh(core_axis_name="c", subcore_axis_name="s", num_cores=NC, num_subcores=NS),
        compiler_params=pltpu.CompilerParams(use_tc_tiling_on_sc=False),
        scratch_types=[
            pltpu.VMEM((K4_INNER, K4_CH), jnp.int32),
            pltpu.VMEM((K4_INNER, K4_CH), jnp.int32),
            pltpu.VMEM((K4_INNER * K4_CH, PPW), jnp.float32),
            pltpu.VMEM((K4_INNER * K4_CH, PPW), jnp.float32),
            pltpu.SemaphoreType.DMA,
        ],
    )(src2d4, dst2d4, pp)

    blk = 10000
    out = pl.pallas_call(
        _edge_body,
        grid=(e // blk,),
        in_specs=[
            pl.BlockSpec((blk, PPW), lambda i: (i, 0)),
            pl.BlockSpec((blk, edge_attr.shape[1]), lambda i: (i, 0)),
            pl.BlockSpec((edge_attr.shape[1], hid), lambda i: (0, 0)),
            pl.BlockSpec((1, hid), lambda i: (0, 0)),
            pl.BlockSpec((hid, out_d), lambda i: (0, 0)),
            pl.BlockSpec((1, out_d), lambda i: (0, 0)),
        ],
        out_specs=pl.BlockSpec((blk, out_d), lambda i: (i, 0)),
        out_shape=jax.ShapeDtypeStruct((e, out_d), jnp.float32),
    )(g, edge_attr, W_edge, b_edge.reshape(1, hid),
      W_cls, b_cls.reshape(1, out_d))

    return out
